# initial kernel scaffold (unmeasured)
import functools

import jax
import jax.numpy as jnp
from jax import lax
from jax.experimental import pallas as pl
from jax.experimental.pallas import tpu as pltpu

N_DEV = 4
SQ = 2048
SKV = 2048
D_MODEL = 1024
H_PER = 8
DH = 128
SCALE = 0.08838834764831843


def _local_partial(x, Wq, K_ext, V_ext, Wo):
    i = lax.axis_index("i")
    xb = x[0].astype(jnp.bfloat16)
    Q = (xb @ Wq.astype(jnp.bfloat16)).reshape(SQ, H_PER, DH)
    K = lax.dynamic_slice_in_dim(K_ext[0], i * H_PER, H_PER, axis=1)
    V = lax.dynamic_slice_in_dim(V_ext[0], i * H_PER, H_PER, axis=1)
    K = K.astype(jnp.bfloat16)
    V = V.astype(jnp.bfloat16)

    qi = jnp.arange(SQ)[:, None]
    ki = jnp.arange(SKV)[None, :]
    mask = (jnp.abs(qi - ki) <= 128) | (ki < 32) | (qi < 32)

    scores = jnp.einsum(
        "ihd,jhd->hij", Q, K, preferred_element_type=jnp.float32
    ) * SCALE
    scores = jnp.where(mask[None, :, :], scores, -1e9)
    m = scores.max(axis=-1, keepdims=True)
    w = jnp.exp(scores - m)
    w = w / w.sum(axis=-1, keepdims=True)
    ctx = jnp.einsum(
        "hij,jhd->ihd", w.astype(jnp.bfloat16), V,
        preferred_element_type=jnp.float32,
    ).reshape(SQ, H_PER * DH).astype(jnp.bfloat16)
    partial = ctx @ Wo.astype(jnp.bfloat16)
    return partial


def _allreduce_body(p_ref, out_ref, comm_ref, send_sems, recv_sems):
    my_pos = lax.axis_index("i")
    left = (my_pos - 1) % N_DEV
    right = (my_pos + 1) % N_DEV

    barrier_sem = pltpu.get_barrier_semaphore()
    for nbr in [left, right]:
        pl.semaphore_signal(
            barrier_sem, inc=1,
            device_id=(nbr,), device_id_type=pl.DeviceIdType.MESH,
        )
    pl.semaphore_wait(barrier_sem, 2)

    out_ref[:, :] = p_ref[:, :].astype(jnp.float32)
    comm_ref[0, :, :] = p_ref[:, :]

    for h in range(N_DEV - 1):
        rdma = pltpu.make_async_remote_copy(
            src_ref=comm_ref.at[h],
            dst_ref=comm_ref.at[h + 1],
            send_sem=send_sems.at[h],
            recv_sem=recv_sems.at[h + 1],
            device_id=(right,),
            device_id_type=pl.DeviceIdType.MESH,
        )
        rdma.start()
        rdma.wait()
        out_ref[:, :] += comm_ref[h + 1, :, :].astype(jnp.float32)


def _allreduce(partial):
    m, n = partial.shape
    return pl.pallas_call(
        _allreduce_body,
        out_shape=jax.ShapeDtypeStruct((m, n), jnp.float32),
        in_specs=[pl.BlockSpec(memory_space=pltpu.VMEM)],
        out_specs=pl.BlockSpec(memory_space=pltpu.VMEM),
        scratch_shapes=[
            pltpu.VMEM((N_DEV, m, n), jnp.bfloat16),
            pltpu.SemaphoreType.DMA((N_DEV,)),
            pltpu.SemaphoreType.DMA((N_DEV,)),
        ],
        compiler_params=pltpu.CompilerParams(collective_id=0),
    )(partial)


def kernel(x, Wq, K_ext, V_ext, Wo):
    partial = _local_partial(x, Wq, K_ext, V_ext, Wo)
    out = _allreduce(partial)
    return out[None, :, :]


# baseline (device time: 158563 ns/iter reference)
import jax
import jax.numpy as jnp
from jax import lax
from jax.experimental import pallas as pl
from jax.experimental.pallas import tpu as pltpu

N_DEV = 4
SQ = 2048
SKV = 2048
D_MODEL = 1024
H_PER = 8
DH = 128
SCALE = 0.08838834764831843
QB = 256
N_QB = SQ // QB
WIN = 512
CH = SQ // N_DEV


def _dot_f32(a, b):
    return lax.dot_general(
        a, b, (((1,), (1,)), ((), ())), preferred_element_type=jnp.float32
    )


def _attn_body(q_ref, k_ref, v_ref, o_ref):
    K = k_ref[0]
    V = v_ref[0]

    for qb in range(N_QB):
        Qb = q_ref[0, qb * QB:(qb + 1) * QB, :]
        qi = qb * QB + jnp.arange(QB)[:, None]
        if qb == 0:
            ki = jnp.arange(SKV)[None, :]
            mask = (jnp.abs(qi - ki) <= 128) | (ki < 32) | (qi < 32)
            S = _dot_f32(Qb, K) * SCALE
            S = jnp.where(mask, S, -1e9)
            m = S.max(axis=-1, keepdims=True)
            w = jnp.exp(S - m)
            w = w / w.sum(axis=-1, keepdims=True)
            ctx = _dot_f32(w.astype(jnp.bfloat16), V.swapaxes(0, 1))
        else:
            start = min(qb * QB - 128, SKV - WIN)
            Kw = K[start:start + WIN]
            Vw = V[start:start + WIN]
            Kg = K[0:128]
            Vg = V[0:128]
            ki1 = start + jnp.arange(WIN)[None, :]
            mask1 = jnp.abs(qi - ki1) <= 128
            mask0 = jnp.arange(128)[None, :] < 32
            S1 = jnp.where(mask1, _dot_f32(Qb, Kw) * SCALE, -1e9)
            S0 = jnp.where(mask0, _dot_f32(Qb, Kg) * SCALE, -1e9)
            m = jnp.maximum(
                S1.max(axis=-1, keepdims=True), S0.max(axis=-1, keepdims=True)
            )
            e1 = jnp.exp(S1 - m)
            e0 = jnp.exp(S0 - m)
            denom = e1.sum(axis=-1, keepdims=True) + e0.sum(axis=-1, keepdims=True)
            ctx = _dot_f32(
                (e1 / denom).astype(jnp.bfloat16), Vw.swapaxes(0, 1)
            ) + _dot_f32((e0 / denom).astype(jnp.bfloat16), Vg.swapaxes(0, 1))
        o_ref[0, qb * QB:(qb + 1) * QB, :] = ctx.astype(jnp.bfloat16)


def _sparse_attn(Qh, Kh, Vh):
    spec = pl.BlockSpec((1, SQ, DH), lambda h: (h, 0, 0))
    return pl.pallas_call(
        _attn_body,
        grid=(H_PER,),
        out_shape=jax.ShapeDtypeStruct((H_PER, SQ, DH), jnp.bfloat16),
        in_specs=[spec, spec, spec],
        out_specs=spec,
    )(Qh, Kh, Vh)


def _ar_body(p_ref, out_ref, rs_send, rs_recv, ag_buf, ag_recv,
             rs_ssem, rs_rsem, ag_ssem, ag_rsem):
    my = lax.axis_index("i")
    left = (my - 1) % N_DEV
    right = (my + 1) % N_DEV

    barrier_sem = pltpu.get_barrier_semaphore()
    for nbr in [left, right]:
        pl.semaphore_signal(
            barrier_sem, inc=1,
            device_id=(nbr,), device_id_type=pl.DeviceIdType.MESH,
        )
    pl.semaphore_wait(barrier_sem, 2)

    for h in range(N_DEV - 1):
        row = ((my - h) % N_DEV) * CH
        if h == 0:
            rs_send[h, :, :] = p_ref[pl.ds(row, CH), :]
        else:
            rs_send[h, :, :] = (
                rs_recv[h - 1, :, :].astype(jnp.float32)
                + p_ref[pl.ds(row, CH), :].astype(jnp.float32)
            ).astype(jnp.bfloat16)
        rdma = pltpu.make_async_remote_copy(
            src_ref=rs_send.at[h],
            dst_ref=rs_recv.at[h],
            send_sem=rs_ssem.at[h],
            recv_sem=rs_rsem.at[h],
            device_id=(right,),
            device_id_type=pl.DeviceIdType.MESH,
        )
        rdma.start()
        rdma.wait()

    own_row = ((my + 1) % N_DEV) * CH
    own = (
        rs_recv[N_DEV - 2, :, :].astype(jnp.float32)
        + p_ref[pl.ds(own_row, CH), :].astype(jnp.float32)
    ).astype(jnp.bfloat16)
    ag_buf[:, :] = own
    out_ref[pl.ds(own_row, CH), :] = own.astype(jnp.float32)

    for g in range(N_DEV - 1):
        src = ag_buf if g == 0 else ag_recv.at[g - 1]
        rdma = pltpu.make_async_remote_copy(
            src_ref=src,
            dst_ref=ag_recv.at[g],
            send_sem=ag_ssem.at[g],
            recv_sem=ag_rsem.at[g],
            device_id=(right,),
            device_id_type=pl.DeviceIdType.MESH,
        )
        rdma.start()
        rdma.wait()
        row = ((my - g) % N_DEV) * CH
        out_ref[pl.ds(row, CH), :] = ag_recv[g, :, :].astype(jnp.float32)


def _allreduce(partial):
    m, n = partial.shape
    return pl.pallas_call(
        _ar_body,
        out_shape=jax.ShapeDtypeStruct((m, n), jnp.float32),
        in_specs=[pl.BlockSpec(memory_space=pltpu.VMEM)],
        out_specs=pl.BlockSpec(memory_space=pltpu.VMEM),
        scratch_shapes=[
            pltpu.VMEM((N_DEV - 1, CH, n), jnp.bfloat16),
            pltpu.VMEM((N_DEV - 1, CH, n), jnp.bfloat16),
            pltpu.VMEM((CH, n), jnp.bfloat16),
            pltpu.VMEM((N_DEV - 1, CH, n), jnp.bfloat16),
            pltpu.SemaphoreType.DMA((N_DEV - 1,)),
            pltpu.SemaphoreType.DMA((N_DEV - 1,)),
            pltpu.SemaphoreType.DMA((N_DEV - 1,)),
            pltpu.SemaphoreType.DMA((N_DEV - 1,)),
        ],
        compiler_params=pltpu.CompilerParams(collective_id=0),
    )(partial)


def kernel(x, Wq, K_ext, V_ext, Wo):
    i = lax.axis_index("i")
    xb = x[0].astype(jnp.bfloat16)
    Q = (xb @ Wq.astype(jnp.bfloat16)).reshape(SQ, H_PER, DH)
    K = lax.dynamic_slice_in_dim(K_ext[0], i * H_PER, H_PER, axis=1)
    V = lax.dynamic_slice_in_dim(V_ext[0], i * H_PER, H_PER, axis=1)
    Qh = Q.transpose(1, 0, 2)
    Kh = K.astype(jnp.bfloat16).transpose(1, 0, 2)
    Vh = V.astype(jnp.bfloat16).transpose(1, 0, 2)

    ctx = _sparse_attn(Qh, Kh, Vh)
    ctx = ctx.transpose(1, 0, 2).reshape(SQ, H_PER * DH)
    partial = ctx @ Wo.astype(jnp.bfloat16)

    out = _allreduce(partial)
    return out[None, :, :]


# device time: 131093 ns/iter; 1.2095x vs baseline; 1.2095x over previous
import jax
import jax.numpy as jnp
from jax import lax
from jax.experimental import pallas as pl
from jax.experimental.pallas import tpu as pltpu

N_DEV = 4
SQ = 2048
SKV = 2048
D_MODEL = 1024
H_PER = 8
DH = 128
SCALE = 0.08838834764831843
QB = 256
WIN = 512
CH = SQ // N_DEV
BF = jnp.bfloat16
F32 = jnp.float32


def _dot_t(a, b):
    return lax.dot_general(
        a, b, (((1,), (1,)), ((), ())), preferred_element_type=F32
    )


def _dot(a, b):
    return lax.dot_general(
        a, b, (((1,), (0,)), ((), ())), preferred_element_type=F32
    )


def _compute_chunk(c, q_ref, k_ref, v_ref, wo_ref, ctx_s, dst):
    rows0 = c * CH
    for h0 in range(H_PER):
        co = h0 * DH
        for sub in range(CH // QB):
            r = rows0 + sub * QB
            Qb = q_ref[pl.ds(pl.multiple_of(r, QB), QB), co:co + DH]
            start = pl.multiple_of(jnp.clip(r - 128, 0, SKV - WIN), 128)
            Kw = k_ref[pl.ds(start, WIN), co:co + DH]
            Vw = v_ref[pl.ds(start, WIN), co:co + DH]
            Kg = k_ref[0:DH, co:co + DH]
            Vg = v_ref[0:DH, co:co + DH]

            qi = r + lax.broadcasted_iota(jnp.int32, (QB, WIN), 0)
            ki1 = start + lax.broadcasted_iota(jnp.int32, (QB, WIN), 1)
            mask1 = (jnp.abs(qi - ki1) <= 128) | (ki1 < 32) | (qi < 32)
            ki0 = lax.broadcasted_iota(jnp.int32, (QB, DH), 1)
            mask0 = (ki0 < 32) & (ki0 < start)

            S1 = jnp.where(mask1, _dot_t(Qb, Kw) * SCALE, -1e9)
            S0 = jnp.where(mask0, _dot_t(Qb, Kg) * SCALE, -1e9)
            m = jnp.maximum(
                S1.max(axis=-1, keepdims=True), S0.max(axis=-1, keepdims=True)
            )
            e1 = jnp.exp(S1 - m)
            e0 = jnp.exp(S0 - m)
            denom = e1.sum(axis=-1, keepdims=True) + e0.sum(
                axis=-1, keepdims=True
            )
            ctx = _dot((e1 / denom).astype(BF), Vw) + _dot(
                (e0 / denom).astype(BF), Vg
            )
            ctx_s[sub * QB:(sub + 1) * QB, co:co + DH] = ctx.astype(BF)

    @pl.when(c == 0)
    def _():
        for h0 in range(H_PER):
            co = h0 * DH
            Qd = q_ref[0:32, co:co + DH]
            Sd = _dot_t(Qd, k_ref[:, co:co + DH]) * SCALE
            md = Sd.max(axis=-1, keepdims=True)
            ed = jnp.exp(Sd - md)
            wd = (ed / ed.sum(axis=-1, keepdims=True)).astype(BF)
            ctx_s[0:32, co:co + DH] = _dot(wd, v_ref[:, co:co + DH]).astype(BF)

    dst[:, :] = _dot(ctx_s[:, :], wo_ref[:, :]).astype(BF)


def _fused_body(q_ref, k_ref, v_ref, wo_ref, out_ref,
                ctx_s, p_s, rs_send, rs_recv, ag_buf, ag_recv,
                rs_ssem, rs_rsem, ag_ssem, ag_rsem):
    my = lax.axis_index("i")
    left = (my - 1) % N_DEV
    right = (my + 1) % N_DEV

    barrier_sem = pltpu.get_barrier_semaphore()
    for nbr in [left, right]:
        pl.semaphore_signal(
            barrier_sem, inc=1,
            device_id=(nbr,), device_id_type=pl.DeviceIdType.MESH,
        )
    pl.semaphore_wait(barrier_sem, 2)

    def rs_rdma(h):
        return pltpu.make_async_remote_copy(
            src_ref=rs_send.at[h],
            dst_ref=rs_recv.at[h],
            send_sem=rs_ssem.at[h],
            recv_sem=rs_rsem.at[h],
            device_id=(right,),
            device_id_type=pl.DeviceIdType.MESH,
        )

    _compute_chunk((my - 0) % N_DEV, q_ref, k_ref, v_ref, wo_ref, ctx_s,
                   rs_send.at[0])
    rdma0 = rs_rdma(0)
    rdma0.start()
    prev = rdma0
    for h in range(1, N_DEV - 1):
        _compute_chunk((my - h) % N_DEV, q_ref, k_ref, v_ref, wo_ref, ctx_s,
                       p_s)
        prev.wait()
        rs_send[h, :, :] = (
            rs_recv[h - 1, :, :].astype(F32) + p_s[:, :].astype(F32)
        ).astype(BF)
        rdma = rs_rdma(h)
        rdma.start()
        prev = rdma

    own_row = ((my + 1) % N_DEV) * CH
    _compute_chunk((my + 1) % N_DEV, q_ref, k_ref, v_ref, wo_ref, ctx_s, p_s)
    prev.wait()
    own = (
        rs_recv[N_DEV - 2, :, :].astype(F32) + p_s[:, :].astype(F32)
    ).astype(BF)
    ag_buf[:, :] = own
    out_ref[pl.ds(pl.multiple_of(own_row, CH), CH), :] = own.astype(F32)

    for g in range(N_DEV - 1):
        src = ag_buf if g == 0 else ag_recv.at[g - 1]
        rdma = pltpu.make_async_remote_copy(
            src_ref=src,
            dst_ref=ag_recv.at[g],
            send_sem=ag_ssem.at[g],
            recv_sem=ag_rsem.at[g],
            device_id=(right,),
            device_id_type=pl.DeviceIdType.MESH,
        )
        rdma.start()
        rdma.wait()
        row = ((my - g) % N_DEV) * CH
        out_ref[pl.ds(pl.multiple_of(row, CH), CH), :] = (
            ag_recv[g, :, :].astype(F32)
        )


def kernel(x, Wq, K_ext, V_ext, Wo):
    i = lax.axis_index("i")
    xb = x[0].astype(BF)
    Qf = xb @ Wq.astype(BF)
    K = lax.dynamic_slice_in_dim(K_ext[0], i * H_PER, H_PER, axis=1)
    V = lax.dynamic_slice_in_dim(V_ext[0], i * H_PER, H_PER, axis=1)
    Kf = K.astype(BF).reshape(SKV, H_PER * DH)
    Vf = V.astype(BF).reshape(SKV, H_PER * DH)
    Wob = Wo.astype(BF)

    out = pl.pallas_call(
        _fused_body,
        out_shape=jax.ShapeDtypeStruct((SQ, D_MODEL), F32),
        in_specs=[pl.BlockSpec(memory_space=pltpu.VMEM)] * 4,
        out_specs=pl.BlockSpec(memory_space=pltpu.VMEM),
        scratch_shapes=[
            pltpu.VMEM((CH, H_PER * DH), BF),
            pltpu.VMEM((CH, D_MODEL), BF),
            pltpu.VMEM((N_DEV - 1, CH, D_MODEL), BF),
            pltpu.VMEM((N_DEV - 1, CH, D_MODEL), BF),
            pltpu.VMEM((CH, D_MODEL), BF),
            pltpu.VMEM((N_DEV - 1, CH, D_MODEL), BF),
            pltpu.SemaphoreType.DMA((N_DEV - 1,)),
            pltpu.SemaphoreType.DMA((N_DEV - 1,)),
            pltpu.SemaphoreType.DMA((N_DEV - 1,)),
            pltpu.SemaphoreType.DMA((N_DEV - 1,)),
        ],
        compiler_params=pltpu.CompilerParams(collective_id=0),
    )(Qf, Kf, Vf, Wob)
    return out[None, :, :]
